# baseline (speedup 1.0000x reference)
import functools

import jax
import jax.numpy as jnp
from jax import lax
from jax.experimental import pallas as pl
from jax.experimental.pallas import tpu as pltpu
from jax.experimental.pallas import tpu_sc as plsc


def _sc_add(x, table, b0, bsc):
    B, S, D = x.shape
    NW = 32
    rows = bsc * S
    rows_per_w = rows // NW
    wpb = S // rows_per_w
    C = 8
    chunks = rows_per_w // C
    NBUF = 4

    mesh = plsc.VectorSubcoreMesh(core_axis_name="c", subcore_axis_name="s")

    @functools.partial(
        pl.kernel, mesh=mesh,
        out_type=jax.ShapeDtypeStruct((bsc, S, D), jnp.float32),
        scratch_types=[
            pltpu.VMEM((NBUF, C, D), jnp.float32),
            pltpu.VMEM((NBUF, C, D), jnp.float32),
            pltpu.VMEM((NBUF, C, D), jnp.float32),
            pltpu.SemaphoreType.DMA((NBUF,)),
            pltpu.SemaphoreType.DMA((NBUF,)),
            pltpu.SemaphoreType.DMA((NBUF,)),
        ],
    )
    def k(x_hbm, t_hbm, out_hbm, xb, tb, ob, sx, st, so):
        w = lax.axis_index("s") * 2 + lax.axis_index("c")
        bw = w // wpb
        row0 = (w % wpb) * rows_per_w

        def in_copies(slot, g):
            r0 = row0 + g * C
            cx = pltpu.make_async_copy(
                x_hbm.at[b0 + bw, pl.ds(r0, C), :], xb.at[slot], sx.at[slot])
            ct = pltpu.make_async_copy(
                t_hbm.at[pl.ds(r0, C), :], tb.at[slot], st.at[slot])
            return cx, ct

        def out_copy(slot, g):
            r0 = row0 + g * C
            return pltpu.make_async_copy(
                ob.at[slot], out_hbm.at[bw, pl.ds(r0, C), :], so.at[slot])

        for b in range(NBUF):
            cx, ct = in_copies(b, b)
            cx.start()
            ct.start()

        def outer(i, _):
            g0 = i * NBUF
            for b in range(NBUF):
                g = g0 + b
                cx, ct = in_copies(b, g)
                cx.wait()
                ct.wait()

                @pl.when(g >= NBUF)
                def _():
                    out_copy(b, g - NBUF).wait()

                vecs_per_row = D // 16

                @plsc.parallel_loop(0, C * vecs_per_row, unroll=8)
                def add_loop(i):
                    r = i // vecs_per_row
                    sl = pl.ds((i % vecs_per_row) * 16, 16)
                    ob[b, r, sl] = xb[b, r, sl] + tb[b, r, sl]

                out_copy(b, g).start()

                @pl.when(g + NBUF < chunks)
                def _():
                    cx2, ct2 = in_copies(b, g + NBUF)
                    cx2.start()
                    ct2.start()
            return 0

        lax.fori_loop(0, chunks // NBUF, outer, 0)
        for b in range(NBUF):
            out_copy(b, chunks - NBUF + b).wait()

    return k(x, table)


def _tc_add(x, table, btc):
    B, S, D = x.shape
    BS = 2048

    def body(x_ref, t_ref, o_ref):
        o_ref[...] = x_ref[...] + t_ref[...]

    return pl.pallas_call(
        body,
        grid=(S // BS, btc),
        in_specs=[
            pl.BlockSpec((1, BS, D), lambda i, b: (b, i, 0)),
            pl.BlockSpec((BS, D), lambda i, b: (i, 0)),
        ],
        out_specs=pl.BlockSpec((1, BS, D), lambda i, b: (b, i, 0)),
        out_shape=jax.ShapeDtypeStruct((btc, S, D), x.dtype),
        compiler_params=pltpu.CompilerParams(
            dimension_semantics=("arbitrary", "arbitrary"),
        ),
    )(x, table)


def kernel(x, table):
    B, S, D = x.shape
    BSC = 1
    sc_out = _sc_add(x, table, B - BSC, BSC)
    tc_out = _tc_add(x, table, B - BSC)
    return jnp.concatenate([tc_out, sc_out], axis=0)


# speedup vs baseline: 1.3198x; 1.3198x over previous
import functools

import jax
import jax.numpy as jnp
from jax import lax
from jax.experimental import pallas as pl
from jax.experimental.pallas import tpu as pltpu
from jax.experimental.pallas import tpu_sc as plsc


def kernel(x, table):
    B, S, D = x.shape
    NW = 32
    rows_per_w = (B * S) // NW
    wpb = S // rows_per_w
    C = 8
    chunks = rows_per_w // C
    NBUF = 4

    mesh = plsc.VectorSubcoreMesh(core_axis_name="c", subcore_axis_name="s")

    @functools.partial(
        pl.kernel, mesh=mesh,
        out_type=jax.ShapeDtypeStruct((B, S, D), jnp.float32),
        scratch_types=[
            pltpu.VMEM((NBUF, C, D), jnp.float32),
            pltpu.VMEM((NBUF, C, D), jnp.float32),
            pltpu.VMEM((NBUF, C, D), jnp.float32),
            pltpu.SemaphoreType.DMA((NBUF,)),
            pltpu.SemaphoreType.DMA((NBUF,)),
            pltpu.SemaphoreType.DMA((NBUF,)),
        ],
    )
    def k(x_hbm, t_hbm, out_hbm, xb, tb, ob, sx, st, so):
        w = lax.axis_index("s") * 2 + lax.axis_index("c")
        bw = w // wpb
        row0 = (w % wpb) * rows_per_w

        def in_copies(slot, g):
            r0 = row0 + g * C
            cx = pltpu.make_async_copy(
                x_hbm.at[bw, pl.ds(r0, C), :], xb.at[slot], sx.at[slot])
            ct = pltpu.make_async_copy(
                t_hbm.at[pl.ds(r0, C), :], tb.at[slot], st.at[slot])
            return cx, ct

        def out_copy(slot, g):
            r0 = row0 + g * C
            return pltpu.make_async_copy(
                ob.at[slot], out_hbm.at[bw, pl.ds(r0, C), :], so.at[slot])

        for b in range(NBUF):
            cx, ct = in_copies(b, b)
            cx.start()
            ct.start()

        def outer(i, _):
            g0 = i * NBUF
            for b in range(NBUF):
                g = g0 + b
                cx, ct = in_copies(b, g)
                cx.wait()
                ct.wait()

                @pl.when(g >= NBUF)
                def _():
                    out_copy(b, g - NBUF).wait()

                vecs_per_row = D // 16

                @plsc.parallel_loop(0, C * vecs_per_row, unroll=8)
                def add_loop(i):
                    r = i // vecs_per_row
                    sl = pl.ds((i % vecs_per_row) * 16, 16)
                    ob[b, r, sl] = xb[b, r, sl] + tb[b, r, sl]
                out_copy(b, g).start()

                @pl.when(g + NBUF < chunks)
                def _():
                    cx2, ct2 = in_copies(b, g + NBUF)
                    cx2.start()
                    ct2.start()
            return 0

        lax.fori_loop(0, chunks // NBUF, outer, 0)
        for b in range(NBUF):
            out_copy(b, chunks - NBUF + b).wait()

    return k(x, table)


# speedup vs baseline: 1.5723x; 1.1913x over previous
import functools

import jax
import jax.numpy as jnp
from jax import lax
from jax.experimental import pallas as pl
from jax.experimental.pallas import tpu as pltpu
from jax.experimental.pallas import tpu_sc as plsc


def kernel(x, table):
    B, S, D = x.shape
    NW = 32
    rows_per_w = (B * S) // NW
    C = 8
    chunks = rows_per_w // C
    GRP = 4
    groups = chunks // GRP

    mesh = plsc.VectorSubcoreMesh(core_axis_name="c", subcore_axis_name="s")

    @functools.partial(
        pl.kernel, mesh=mesh,
        out_type=jax.ShapeDtypeStruct((B, S, D), jnp.float32),
        scratch_types=[
            pltpu.VMEM((GRP, C, D), jnp.float32),
            pltpu.VMEM((GRP, C, D), jnp.float32),
            pltpu.VMEM((GRP, C, D), jnp.float32),
            pltpu.VMEM_SHARED((4, 2 * GRP, C, D), jnp.float32),
            pltpu.SemaphoreType.DMA((GRP,)),
            pltpu.SemaphoreType.DMA((GRP,)),
            pltpu.SemaphoreType.DMA((GRP,)),
            pltpu.SemaphoreType.DMA((GRP,)),
        ],
    )
    def k(x_hbm, t_hbm, out_hbm, xb, tb, ob, tsh, sx, st, so, sl):
        cid = lax.axis_index("c")
        sid = lax.axis_index("s")
        bw = sid // 4
        slab = sid % 4
        row0 = (cid * 4 + slab) * rows_per_w

        def x_copy(j, g):
            r0 = row0 + g * C
            return pltpu.make_async_copy(
                x_hbm.at[bw, pl.ds(r0, C), :], xb.at[j], sx.at[j])

        def tload_copy(k_, g, half):
            r0 = row0 + g * C
            return pltpu.make_async_copy(
                t_hbm.at[pl.ds(r0, C), :], tsh.at[slab, half * GRP + k_],
                sl.at[k_])

        def tcons_copy(j, half):
            return pltpu.make_async_copy(
                tsh.at[slab, half * GRP + j], tb.at[j], st.at[j])

        def out_copy(j, g):
            r0 = row0 + g * C
            return pltpu.make_async_copy(
                ob.at[j], out_hbm.at[bw, pl.ds(r0, C), :], so.at[j])

        for j in range(GRP):
            x_copy(j, j).start()

        @pl.when(bw == 0)
        def _():
            for k_ in range(GRP):
                tload_copy(k_, k_, 0).start()

        def outer(G, _):
            half = G % 2
            is_loader = bw == (G % 4)
            is_next_loader = bw == ((G + 1) % 4)

            @pl.when(is_loader)
            def _():
                for k_ in range(GRP):
                    tload_copy(k_, G * GRP + k_, half).wait()

            plsc.subcore_barrier()

            @pl.when(is_next_loader & (G + 1 < groups))
            def _():
                for k_ in range(GRP):
                    tload_copy(k_, (G + 1) * GRP + k_, 1 - half).start()

            for j in range(GRP):
                tcons_copy(j, half).start()

            for j in range(GRP):
                g = G * GRP + j
                tcons_copy(j, half).wait()
                x_copy(j, g).wait()

                @pl.when(g >= GRP)
                def _():
                    out_copy(j, g - GRP).wait()

                vecs_per_row = D // 16

                @plsc.parallel_loop(0, C * vecs_per_row, unroll=8)
                def add_loop(i):
                    r = i // vecs_per_row
                    sl_ = pl.ds((i % vecs_per_row) * 16, 16)
                    ob[j, r, sl_] = xb[j, r, sl_] + tb[j, r, sl_]
                out_copy(j, g).start()

                @pl.when(g + GRP < chunks)
                def _():
                    x_copy(j, g + GRP).start()
            return 0

        lax.fori_loop(0, groups, outer, 0)
        for j in range(GRP):
            out_copy(j, chunks - GRP + j).wait()

    return k(x, table)
